# Initial kernel scaffold; baseline (speedup 1.0000x reference)
import jax
import jax.numpy as jnp
from jax.experimental import pallas as pl


def kernel(X, emb, Wh, bh, Wo, bo):
    raise NotImplementedError("write your pallas kernel here")


# speedup vs baseline: 1.5111x; 1.5111x over previous
import functools

import jax
import jax.numpy as jnp
from jax import lax
from jax.experimental import pallas as pl
from jax.experimental.pallas import tpu as pltpu
from jax.experimental.pallas import tpu_sc as plsc

SEQ = 16384
DIM = 128
NC = 2
NS = 16
NW = NC * NS
B_PER_W = SEQ // NW
CHUNK = 128
NCHUNK = B_PER_W // CHUNK


def _sc_gather_partial_sums(x3, emb):
  mesh = plsc.VectorSubcoreMesh(core_axis_name="c", subcore_axis_name="s")

  @functools.partial(
      pl.kernel,
      mesh=mesh,
      out_type=jax.ShapeDtypeStruct((NW, DIM), jnp.float32),
      scratch_types=[
          pltpu.VMEM((NCHUNK, CHUNK), jnp.int32),
          pltpu.VMEM((NCHUNK * CHUNK, DIM), jnp.float32),
          pltpu.VMEM((DIM,), jnp.float32),
          pltpu.SemaphoreType.DMA,
      ],
  )
  def k(x_hbm, emb_hbm, out_hbm, idx_v, rows_v, acc_v, sem):
    wid = lax.axis_index("s") * NC + lax.axis_index("c")
    pltpu.sync_copy(x_hbm.at[wid], idx_v)
    copies = []
    for j in range(NCHUNK):
      copies.append(
          pltpu.async_copy(
              emb_hbm.at[idx_v.at[j]],
              rows_v.at[pl.ds(j * CHUNK, CHUNK)],
              sem,
          )
      )
    for c in copies:
      c.wait()

    nv = DIM // 16

    def body(i, carry):
      return tuple(carry[d] + rows_v[i, pl.ds(d * 16, 16)] for d in range(nv))

    acc = lax.fori_loop(
        0, B_PER_W, body,
        tuple(jnp.zeros((16,), jnp.float32) for _ in range(nv)),
    )
    for d in range(nv):
      acc_v[pl.ds(d * 16, 16)] = acc[d]
    pltpu.sync_copy(acc_v, out_hbm.at[wid])

  return k(x3, emb)


def _tc_mlp(partials, wh, bh2, wo, bo2):
  def body(p_ref, wh_ref, bh_ref, wo_ref, bo_ref, o_ref):
    embed = jnp.sum(p_ref[...], axis=0, keepdims=True) * (1.0 / SEQ)
    h = jax.lax.dot_general(
        embed, wh_ref[...], (((1,), (1,)), ((), ())),
        preferred_element_type=jnp.float32) + bh_ref[...]
    h = jnp.tanh(h)
    o = jax.lax.dot_general(
        h, wo_ref[...], (((1,), (1,)), ((), ())),
        preferred_element_type=jnp.float32) + bo_ref[...]
    m = jnp.max(o, axis=1, keepdims=True)
    e = jnp.exp(o - m)
    o_ref[...] = e / jnp.sum(e, axis=1, keepdims=True)

  return pl.pallas_call(
      body,
      out_shape=jax.ShapeDtypeStruct((1, 20), jnp.float32),
  )(partials, wh, bh2, wo, bo2)


@jax.jit
def kernel(X, emb, Wh, bh, Wo, bo):
  x3 = X.astype(jnp.int32).reshape(NW, NCHUNK, CHUNK)
  partials = _sc_gather_partial_sums(x3, emb)
  out = _tc_mlp(partials, Wh, bh.reshape(1, -1), Wo, bo.reshape(1, -1))
  return out.reshape(20)


# speedup vs baseline: 1.5209x; 1.0065x over previous
import functools

import jax
import jax.numpy as jnp
from jax import lax
from jax.experimental import pallas as pl
from jax.experimental.pallas import tpu as pltpu
from jax.experimental.pallas import tpu_sc as plsc

SEQ = 16384
DIM = 128
NC = 2
NS = 16
NW = NC * NS
B_PER_W = SEQ // NW
CHUNK = 128
NCHUNK = B_PER_W // CHUNK


def _sc_gather_partial_sums(x3, emb):
  mesh = plsc.VectorSubcoreMesh(core_axis_name="c", subcore_axis_name="s")

  nv = DIM // 16

  @functools.partial(
      pl.kernel,
      mesh=mesh,
      out_type=jax.ShapeDtypeStruct((NW, DIM), jnp.float32),
      scratch_types=[
          pltpu.VMEM((NCHUNK, CHUNK), jnp.int32),
          pltpu.VMEM((CHUNK, DIM), jnp.float32),
          pltpu.VMEM((CHUNK, DIM), jnp.float32),
          pltpu.VMEM((DIM,), jnp.float32),
          pltpu.SemaphoreType.DMA,
          pltpu.SemaphoreType.DMA,
      ],
  )
  def k(x_hbm, emb_hbm, out_hbm, idx_v, rows0, rows1, acc_v, sem0, sem1):
    wid = lax.axis_index("s") * NC + lax.axis_index("c")
    pltpu.sync_copy(x_hbm.at[wid], idx_v)
    bufs = (rows0, rows1)
    sems = (sem0, sem1)
    inflight = [
        pltpu.async_copy(emb_hbm.at[idx_v.at[j]], bufs[j % 2], sems[j % 2])
        for j in range(2)
    ]

    def accumulate(buf, acc):
      @plsc.parallel_loop(0, CHUNK, unroll=4, carry=acc)
      def final(i, c):
        return tuple(c[d] + buf[i, pl.ds(d * 16, 16)] for d in range(nv))

      return final

    acc = tuple(jnp.zeros((16,), jnp.float32) for _ in range(nv))
    for j in range(NCHUNK):
      inflight[j % 2].wait()
      acc = accumulate(bufs[j % 2], acc)
      if j + 2 < NCHUNK:
        inflight[j % 2] = pltpu.async_copy(
            emb_hbm.at[idx_v.at[j + 2]], bufs[j % 2], sems[j % 2])
    for d in range(nv):
      acc_v[pl.ds(d * 16, 16)] = acc[d]
    pltpu.sync_copy(acc_v, out_hbm.at[wid])

  return k(x3, emb)


def _tc_mlp(partials, wh, bh2, wo, bo2):
  def body(p_ref, wh_ref, bh_ref, wo_ref, bo_ref, o_ref):
    embed = jnp.sum(p_ref[...], axis=0, keepdims=True) * (1.0 / SEQ)
    h = jax.lax.dot_general(
        embed, wh_ref[...], (((1,), (1,)), ((), ())),
        preferred_element_type=jnp.float32) + bh_ref[...]
    h = jnp.tanh(h)
    o = jax.lax.dot_general(
        h, wo_ref[...], (((1,), (1,)), ((), ())),
        preferred_element_type=jnp.float32) + bo_ref[...]
    m = jnp.max(o, axis=1, keepdims=True)
    e = jnp.exp(o - m)
    o_ref[...] = e / jnp.sum(e, axis=1, keepdims=True)

  return pl.pallas_call(
      body,
      out_shape=jax.ShapeDtypeStruct((1, 20), jnp.float32),
  )(partials, wh, bh2, wo, bo2)


@jax.jit
def kernel(X, emb, Wh, bh, Wo, bo):
  x3 = X.astype(jnp.int32).reshape(NW, NCHUNK, CHUNK)
  partials = _sc_gather_partial_sums(x3, emb)
  out = _tc_mlp(partials, Wh, bh.reshape(1, -1), Wo, bo.reshape(1, -1))
  return out.reshape(20)
